# baseline (speedup 1.0000x reference)
import jax
import jax.numpy as jnp
from jax.experimental import pallas as pl
from jax.experimental.pallas import tpu as pltpu

_SLOPE = 0.01


def _fused_forward(x, eps, p, block_b=256):
    B, T, f_total = x.shape
    latent = eps.shape[2]
    H1 = p["whh1"].shape[0]
    H2 = p["whh2"].shape[0]
    nb_features = p["we1"].shape[0] - H2
    d_in = f_total - nb_features
    dec_out = p["wdo"].shape[1]

    bb = min(block_b, B)
    assert B % bb == 0

    def body(x_ref, eps_ref, fcw_ref, fcb_ref,
             wih1_ref, whh1_ref, b1_ref, wih2_ref, whh2_ref, b2_ref,
             we1_ref, be1_ref, we2_ref, be2_ref, wmv_ref, bmv_ref,
             wd1_ref, bd1_ref, wd2_ref, bd2_ref, wdo_ref, bdo_ref,
             xrec_ref, mean_ref, logvar_ref, z_ref, hs_ref):
        f32 = jnp.float32
        M = T * bb

        def leaky(v):
            return jnp.maximum(v, _SLOPE * v)

        bf16 = jnp.bfloat16

        def mm(a, w):
            return jnp.dot(a.astype(bf16), w.astype(bf16),
                           preferred_element_type=f32)

        xf = x_ref[...].astype(bf16)
        ef = eps_ref[...]
        xx = jnp.concatenate(
            [xf[:, t * f_total:(t + 1) * f_total] for t in range(T)], axis=0)
        xin = xx[:, :d_in]
        xtail = xx[:, d_in:]

        fc = leaky(mm(xin, fcw_ref[...]) + fcb_ref[...]).astype(bf16)

        w1w = jnp.concatenate([wih1_ref[...], whh1_ref[...]], axis=0)
        b1v = b1_ref[...]
        w2w = jnp.concatenate([wih2_ref[...], whh2_ref[...]], axis=0)
        b2v = b2_ref[...]

        def gates(g, h_prev_c, H):
            s = jax.nn.sigmoid(g)
            gg = jnp.tanh(g[:, 2 * H:3 * H])
            c = s[:, H:2 * H] * h_prev_c + s[:, 0:H] * gg
            h = (s[:, 3 * H:4 * H] * jnp.tanh(c)).astype(bf16)
            return h, c

        def lstm_step(inp, h, c, w, bv, H):
            return gates(mm(jnp.concatenate([inp, h], axis=1), w) + bv, c, H)

        nch = max(1, bb // 256)
        hb = bb // nch
        state = [[jnp.zeros((hb, H1), bf16), jnp.zeros((hb, H1), f32),
                  jnp.zeros((hb, H2), bf16), jnp.zeros((hb, H2), f32)]
                 for _ in range(nch)]
        for t in range(T):
            for k in range(nch):
                h1, c1, h2, c2 = state[k]
                lo = t * bb + k * hb
                h1, c1 = lstm_step(fc[lo:lo + hb, :], h1, c1, w1w, b1v, H1)
                h2, c2 = lstm_step(h1, h2, c2, w2w, b2v, H2)
                state[k] = [h1, c1, h2, c2]
                hs_ref[lo:lo + hb, :] = h2
        hseq = hs_ref[...]

        def to_flat(v):
            return jnp.concatenate(
                [v[t * bb:(t + 1) * bb, :] for t in range(T)], axis=1)

        eh = leaky(mm(jnp.concatenate([hseq, xtail], axis=1), we1_ref[...])
                   + be1_ref[...]).astype(bf16)
        eh = leaky(mm(eh, we2_ref[...]) + be2_ref[...]).astype(bf16)
        mv = mm(eh, wmv_ref[...]) + bmv_ref[...]
        meanf = to_flat(mv[:, :latent])
        logvarf = to_flat(mv[:, latent:])
        zf = meanf + jnp.exp(0.5 * logvarf) * ef
        z_tm = jnp.concatenate(
            [zf[:, t * latent:(t + 1) * latent] for t in range(T)],
            axis=0).astype(bf16)

        dh = leaky(mm(jnp.concatenate([hseq, z_tm], axis=1),
                      wd1_ref[...]) + bd1_ref[...]).astype(bf16)
        dh = leaky(mm(dh, wd2_ref[...]) + bd2_ref[...]).astype(bf16)
        xrec = jax.nn.sigmoid(mm(dh, wdo_ref[...]) + bdo_ref[...])

        xrec_ref[...] = to_flat(xrec)
        mean_ref[...] = meanf
        logvar_ref[...] = logvarf
        z_ref[...] = zf

    def full(a):
        return pl.BlockSpec(a.shape, lambda i, _n=a.ndim: (0,) * _n)

    weights = (p["fc_w"], p["fc_b"],
               p["wih1"], p["whh1"], p["b1"], p["wih2"], p["whh2"], p["b2"],
               p["we1"], p["be1"], p["we2"], p["be2"], p["wmv"], p["bmv"],
               p["wd1"], p["bd1"], p["wd2"], p["bd2"], p["wdo"], p["bdo"])

    macs = (d_in * H1 + (H1 + H1) * 4 * H1 + (H1 + H2) * 4 * H2
            + (H2 + nb_features) * p["we1"].shape[1]
            + p["we2"].shape[0] * p["we2"].shape[1]
            + p["wmv"].shape[0] * 2 * latent
            + (H2 + latent) * p["wd1"].shape[1]
            + p["wd2"].shape[0] * p["wd2"].shape[1]
            + p["wdo"].shape[0] * dec_out)
    cost = pl.CostEstimate(
        flops=2 * B * T * macs,
        transcendentals=B * T * (6 * H1 + 6 * H2 + latent + dec_out),
        bytes_accessed=4 * (x.size + eps.size
                            + B * T * (dec_out + 3 * latent)
                            + sum(wt.size for wt in weights)),
    )

    out = pl.pallas_call(
        body,
        grid=(B // bb,),
        in_specs=[
            pl.BlockSpec((bb, T * f_total), lambda i: (i, 0)),
            pl.BlockSpec((bb, T * latent), lambda i: (i, 0)),
        ] + [full(wt) for wt in weights],
        out_specs=[
            pl.BlockSpec((bb, T * dec_out), lambda i: (i, 0)),
            pl.BlockSpec((bb, T * latent), lambda i: (i, 0)),
            pl.BlockSpec((bb, T * latent), lambda i: (i, 0)),
            pl.BlockSpec((bb, T * latent), lambda i: (i, 0)),
        ],
        out_shape=[
            jax.ShapeDtypeStruct((B, T * dec_out), jnp.float32),
            jax.ShapeDtypeStruct((B, T * latent), jnp.float32),
            jax.ShapeDtypeStruct((B, T * latent), jnp.float32),
            jax.ShapeDtypeStruct((B, T * latent), jnp.float32),
        ],
        scratch_shapes=[pltpu.VMEM((T * bb, H2), jnp.bfloat16)],
        compiler_params=pltpu.CompilerParams(
            dimension_semantics=("parallel",)),
        cost_estimate=cost,
    )(x.reshape(B, T * f_total), eps.reshape(B, T * latent), *weights)

    xrec, mean, logvar, z = out
    return (xrec.reshape(B, T, dec_out),
            mean.reshape(B, T, latent),
            logvar.reshape(B, T, latent),
            z.reshape(B, T, latent))


def kernel(x, eps, fc_w, fc_b, wih1, whh1, wih2, whh2, we1, we2, wmv,
           wd1, wd2, wdo, b1, b2, be1, be2, bmv, bd1, bd2, bdo):
    p = {"fc_w": fc_w, "fc_b": fc_b,
         "wih1": wih1, "whh1": whh1, "wih2": wih2, "whh2": whh2,
         "we1": we1, "we2": we2, "wmv": wmv,
         "wd1": wd1, "wd2": wd2, "wdo": wdo,
         "b1": b1, "b2": b2, "be1": be1, "be2": be2,
         "bmv": bmv, "bd1": bd1, "bd2": bd2, "bdo": bdo}
    return _fused_forward(x, eps, p, block_b=512)


# speedup vs baseline: 1.0149x; 1.0149x over previous
import jax
import jax.numpy as jnp
from jax.experimental import pallas as pl
from jax.experimental.pallas import tpu as pltpu

_SLOPE = 0.01


def _fused_forward(x, eps, p, block_b=256):
    B, T, f_total = x.shape
    latent = eps.shape[2]
    H1 = p["whh1"].shape[0]
    H2 = p["whh2"].shape[0]
    nb_features = p["we1"].shape[0] - H2
    d_in = f_total - nb_features
    dec_out = p["wdo"].shape[1]

    bb = min(block_b, B)
    assert B % bb == 0

    def body(x_ref, eps_ref, fcw_ref, fcb_ref,
             wih1_ref, whh1_ref, b1_ref, wih2_ref, whh2_ref, b2_ref,
             we1_ref, be1_ref, we2_ref, be2_ref, wmv_ref, bmv_ref,
             wd1_ref, bd1_ref, wd2_ref, bd2_ref, wdo_ref, bdo_ref,
             xrec_ref, mean_ref, logvar_ref, z_ref):
        f32 = jnp.float32
        M = T * bb

        def leaky(v):
            return jnp.maximum(v, _SLOPE * v)

        bf16 = jnp.bfloat16

        def mm(a, w):
            return jnp.dot(a.astype(bf16), w.astype(bf16),
                           preferred_element_type=f32)

        xf = x_ref[...].astype(bf16)
        ef = eps_ref[...]
        xx = jnp.concatenate(
            [xf[:, t * f_total:(t + 1) * f_total] for t in range(T)], axis=0)
        xin = xx[:, :d_in]
        xtail = xx[:, d_in:]

        fc = leaky(mm(xin, fcw_ref[...]) + fcb_ref[...]).astype(bf16)

        w1w = jnp.concatenate([wih1_ref[...], whh1_ref[...]], axis=0)
        b1v = b1_ref[...]
        w2w = jnp.concatenate([wih2_ref[...], whh2_ref[...]], axis=0)
        b2v = b2_ref[...]

        def gates(g, h_prev_c, H):
            s = jax.nn.sigmoid(g)
            gg = jnp.tanh(g[:, 2 * H:3 * H])
            c = s[:, H:2 * H] * h_prev_c + s[:, 0:H] * gg
            h = (s[:, 3 * H:4 * H] * jnp.tanh(c)).astype(bf16)
            return h, c

        def lstm_step(inp, h, c, w, bv, H):
            return gates(mm(jnp.concatenate([inp, h], axis=1), w) + bv, c, H)

        nch = max(1, bb // 256)
        hb = bb // nch
        state = [[jnp.zeros((hb, H1), bf16), jnp.zeros((hb, H1), f32),
                  jnp.zeros((hb, H2), bf16), jnp.zeros((hb, H2), f32)]
                 for _ in range(nch)]
        rows = []
        for t in range(T):
            for k in range(nch):
                h1, c1, h2, c2 = state[k]
                lo = t * bb + k * hb
                h1, c1 = lstm_step(fc[lo:lo + hb, :], h1, c1, w1w, b1v, H1)
                h2, c2 = lstm_step(h1, h2, c2, w2w, b2v, H2)
                state[k] = [h1, c1, h2, c2]
                rows.append(h2)
        hseq = jnp.concatenate(rows, axis=0)

        def to_flat(v):
            return jnp.concatenate(
                [v[t * bb:(t + 1) * bb, :] for t in range(T)], axis=1)

        eh = leaky(mm(jnp.concatenate([hseq, xtail], axis=1), we1_ref[...])
                   + be1_ref[...]).astype(bf16)
        eh = leaky(mm(eh, we2_ref[...]) + be2_ref[...]).astype(bf16)
        mv = mm(eh, wmv_ref[...]) + bmv_ref[...]
        meanf = to_flat(mv[:, :latent])
        logvarf = to_flat(mv[:, latent:])
        zf = meanf + jnp.exp(0.5 * logvarf) * ef
        z_tm = jnp.concatenate(
            [zf[:, t * latent:(t + 1) * latent] for t in range(T)],
            axis=0).astype(bf16)

        dh = leaky(mm(jnp.concatenate([hseq, z_tm], axis=1),
                      wd1_ref[...]) + bd1_ref[...]).astype(bf16)
        dh = leaky(mm(dh, wd2_ref[...]) + bd2_ref[...]).astype(bf16)
        xrec = jax.nn.sigmoid(mm(dh, wdo_ref[...]) + bdo_ref[...])

        xrec_ref[...] = to_flat(xrec)
        mean_ref[...] = meanf
        logvar_ref[...] = logvarf
        z_ref[...] = zf

    def full(a):
        return pl.BlockSpec(a.shape, lambda i, _n=a.ndim: (0,) * _n)

    weights = (p["fc_w"], p["fc_b"],
               p["wih1"], p["whh1"], p["b1"], p["wih2"], p["whh2"], p["b2"],
               p["we1"], p["be1"], p["we2"], p["be2"], p["wmv"], p["bmv"],
               p["wd1"], p["bd1"], p["wd2"], p["bd2"], p["wdo"], p["bdo"])

    macs = (d_in * H1 + (H1 + H1) * 4 * H1 + (H1 + H2) * 4 * H2
            + (H2 + nb_features) * p["we1"].shape[1]
            + p["we2"].shape[0] * p["we2"].shape[1]
            + p["wmv"].shape[0] * 2 * latent
            + (H2 + latent) * p["wd1"].shape[1]
            + p["wd2"].shape[0] * p["wd2"].shape[1]
            + p["wdo"].shape[0] * dec_out)
    cost = pl.CostEstimate(
        flops=2 * B * T * macs,
        transcendentals=B * T * (6 * H1 + 6 * H2 + latent + dec_out),
        bytes_accessed=4 * (x.size + eps.size
                            + B * T * (dec_out + 3 * latent)
                            + sum(wt.size for wt in weights)),
    )

    out = pl.pallas_call(
        body,
        grid=(B // bb,),
        in_specs=[
            pl.BlockSpec((bb, T * f_total), lambda i: (i, 0)),
            pl.BlockSpec((bb, T * latent), lambda i: (i, 0)),
        ] + [full(wt) for wt in weights],
        out_specs=[
            pl.BlockSpec((bb, T * dec_out), lambda i: (i, 0)),
            pl.BlockSpec((bb, T * latent), lambda i: (i, 0)),
            pl.BlockSpec((bb, T * latent), lambda i: (i, 0)),
            pl.BlockSpec((bb, T * latent), lambda i: (i, 0)),
        ],
        out_shape=[
            jax.ShapeDtypeStruct((B, T * dec_out), jnp.float32),
            jax.ShapeDtypeStruct((B, T * latent), jnp.float32),
            jax.ShapeDtypeStruct((B, T * latent), jnp.float32),
            jax.ShapeDtypeStruct((B, T * latent), jnp.float32),
        ],
        compiler_params=pltpu.CompilerParams(
            dimension_semantics=("parallel",)),
        cost_estimate=cost,
    )(x.reshape(B, T * f_total), eps.reshape(B, T * latent), *weights)

    xrec, mean, logvar, z = out
    return (xrec.reshape(B, T, dec_out),
            mean.reshape(B, T, latent),
            logvar.reshape(B, T, latent),
            z.reshape(B, T, latent))


def kernel(x, eps, fc_w, fc_b, wih1, whh1, wih2, whh2, we1, we2, wmv,
           wd1, wd2, wdo, b1, b2, be1, be2, bmv, bd1, bd2, bdo):
    p = {"fc_w": fc_w, "fc_b": fc_b,
         "wih1": wih1, "whh1": whh1, "wih2": wih2, "whh2": whh2,
         "we1": we1, "we2": we2, "wmv": wmv,
         "wd1": wd1, "wd2": wd2, "wdo": wdo,
         "b1": b1, "b2": b2, "be1": be1, "be2": be2,
         "bmv": bmv, "bd1": bd1, "bd2": bd2, "bdo": bdo}
    return _fused_forward(x, eps, p, block_b=512)
